# baseline (device time: 168204 ns/iter reference)
import jax
import jax.numpy as jnp
from jax import lax
from jax.experimental import pallas as pl
from jax.experimental.pallas import tpu as pltpu

N_Z = 4


def kernel(dy, W):
    m, k = dy.shape
    n, _ = W.shape

    def body(dy_ref, w_ref, out_ref, comm_ref, send_sems, recv_sems):
        my_x = lax.axis_index("x")
        my_y = lax.axis_index("y")
        my_z = lax.axis_index("z")
        left = (my_z - 1) % N_Z
        right = (my_z + 1) % N_Z

        barrier_sem = pltpu.get_barrier_semaphore()
        for nbr in (left, right):
            pl.semaphore_signal(
                barrier_sem,
                inc=1,
                device_id=(my_x, my_y, nbr),
                device_id_type=pl.DeviceIdType.MESH,
            )
        pl.semaphore_wait(barrier_sem, 2)

        partial = lax.dot_general(
            dy_ref[...],
            w_ref[...],
            dimension_numbers=(((1,), (1,)), ((), ())),
            preferred_element_type=jnp.float32,
        )
        out_ref[...] = partial
        comm_ref[0] = partial

        for h in range(N_Z - 1):
            rdma = pltpu.make_async_remote_copy(
                src_ref=comm_ref.at[h],
                dst_ref=comm_ref.at[h + 1],
                send_sem=send_sems.at[h],
                recv_sem=recv_sems.at[h + 1],
                device_id=(my_x, my_y, right),
                device_id_type=pl.DeviceIdType.MESH,
            )
            rdma.start()
            rdma.wait()
            out_ref[...] += comm_ref[h + 1]

    return pl.pallas_call(
        body,
        out_shape=jax.ShapeDtypeStruct((m, n), jnp.float32),
        in_specs=[
            pl.BlockSpec(memory_space=pltpu.VMEM),
            pl.BlockSpec(memory_space=pltpu.VMEM),
        ],
        out_specs=pl.BlockSpec(memory_space=pltpu.VMEM),
        scratch_shapes=[
            pltpu.VMEM((N_Z, m, n), jnp.float32),
            pltpu.SemaphoreType.DMA((N_Z,)),
            pltpu.SemaphoreType.DMA((N_Z,)),
        ],
        compiler_params=pltpu.CompilerParams(collective_id=0),
    )(dy, W)


# device time: 105433 ns/iter; 1.5954x vs baseline; 1.5954x over previous
import jax
import jax.numpy as jnp
from jax import lax
from jax.experimental import pallas as pl
from jax.experimental.pallas import tpu as pltpu

N_Z = 4
CHUNK = 128


def kernel(dy, W):
    m, k = dy.shape
    n, _ = W.shape

    def body(dy_ref, w_ref, out_ref, rs_buf, rs_send, rs_recv, ag_send, ag_recv):
        my_x = lax.axis_index("x")
        my_y = lax.axis_index("y")
        my_z = lax.axis_index("z")
        left = (my_z - 1) % N_Z
        right = (my_z + 1) % N_Z

        barrier_sem = pltpu.get_barrier_semaphore()
        for nbr in (left, right):
            pl.semaphore_signal(
                barrier_sem,
                inc=1,
                device_id=(my_x, my_y, nbr),
                device_id_type=pl.DeviceIdType.MESH,
            )
        pl.semaphore_wait(barrier_sem, 2)

        out_ref[...] = lax.dot_general(
            dy_ref[...],
            w_ref[...],
            dimension_numbers=(((1,), (1,)), ((), ())),
            preferred_element_type=jnp.float32,
        )

        def nbr_of(d):
            return right if d == 0 else left

        for s in range(N_Z - 1):
            rdmas = []
            for d in range(2):
                send_c = (my_z - s) % N_Z if d == 0 else (my_z + s) % N_Z
                row0 = d * (N_Z * CHUNK) + send_c * CHUNK
                r = pltpu.make_async_remote_copy(
                    src_ref=out_ref.at[pl.ds(row0, CHUNK), :],
                    dst_ref=rs_buf.at[d, s],
                    send_sem=rs_send.at[d, s],
                    recv_sem=rs_recv.at[d, s],
                    device_id=(my_x, my_y, nbr_of(d)),
                    device_id_type=pl.DeviceIdType.MESH,
                )
                r.start()
                rdmas.append(r)
            for d, r in enumerate(rdmas):
                r.wait()
                recv_c = (my_z - s - 1) % N_Z if d == 0 else (my_z + s + 1) % N_Z
                row0 = d * (N_Z * CHUNK) + recv_c * CHUNK
                out_ref[pl.ds(row0, CHUNK), :] += rs_buf[d, s]

        for s in range(N_Z - 1):
            rdmas = []
            for d in range(2):
                send_c = (my_z + 1 - s) % N_Z if d == 0 else (my_z - 1 + s) % N_Z
                row0 = d * (N_Z * CHUNK) + send_c * CHUNK
                r = pltpu.make_async_remote_copy(
                    src_ref=out_ref.at[pl.ds(row0, CHUNK), :],
                    dst_ref=out_ref.at[pl.ds(row0, CHUNK), :],
                    send_sem=ag_send.at[d, s],
                    recv_sem=ag_recv.at[d, s],
                    device_id=(my_x, my_y, nbr_of(d)),
                    device_id_type=pl.DeviceIdType.MESH,
                )
                r.start()
                rdmas.append(r)
            for r in rdmas:
                r.wait()

    return pl.pallas_call(
        body,
        out_shape=jax.ShapeDtypeStruct((m, n), jnp.float32),
        in_specs=[
            pl.BlockSpec(memory_space=pltpu.VMEM),
            pl.BlockSpec(memory_space=pltpu.VMEM),
        ],
        out_specs=pl.BlockSpec(memory_space=pltpu.VMEM),
        scratch_shapes=[
            pltpu.VMEM((2, N_Z - 1, CHUNK, n), jnp.float32),
            pltpu.SemaphoreType.DMA((2, N_Z - 1)),
            pltpu.SemaphoreType.DMA((2, N_Z - 1)),
            pltpu.SemaphoreType.DMA((2, N_Z - 1)),
            pltpu.SemaphoreType.DMA((2, N_Z - 1)),
        ],
        compiler_params=pltpu.CompilerParams(collective_id=0),
    )(dy, W)


# device time: 101680 ns/iter; 1.6542x vs baseline; 1.0369x over previous
import jax
import jax.numpy as jnp
from jax import lax
from jax.experimental import pallas as pl
from jax.experimental.pallas import tpu as pltpu

N_Z = 4
CHUNK = 128


def kernel(dy, W):
    m, k = dy.shape
    n, _ = W.shape

    def body(dy_ref, w_ref, out_ref, rs_buf, rs_send, rs_recv, ag_send, ag_recv):
        my_x = lax.axis_index("x")
        my_y = lax.axis_index("y")
        my_z = lax.axis_index("z")
        left = (my_z - 1) % N_Z
        right = (my_z + 1) % N_Z

        def nbr_of(d):
            return right if d == 0 else left

        def chunk_of(d, j):
            return (my_z - j) % N_Z if d == 0 else (my_z + j) % N_Z

        def row0(d, c):
            return d * (N_Z * CHUNK) + c * CHUNK

        def compute(d, j):
            r0 = row0(d, chunk_of(d, j))
            out_ref[pl.ds(r0, CHUNK), :] = lax.dot_general(
                dy_ref[pl.ds(r0, CHUNK), :],
                w_ref[...],
                dimension_numbers=(((1,), (1,)), ((), ())),
                preferred_element_type=jnp.float32,
            )

        def rs_rdma(d, s):
            r0 = row0(d, chunk_of(d, s))
            return pltpu.make_async_remote_copy(
                src_ref=out_ref.at[pl.ds(r0, CHUNK), :],
                dst_ref=rs_buf.at[d, s],
                send_sem=rs_send.at[d, s],
                recv_sem=rs_recv.at[d, s],
                device_id=(my_x, my_y, nbr_of(d)),
                device_id_type=pl.DeviceIdType.MESH,
            )

        for d in (0, 1):
            compute(d, 0)

        barrier_sem = pltpu.get_barrier_semaphore()
        for nbr in (left, right):
            pl.semaphore_signal(
                barrier_sem,
                inc=1,
                device_id=(my_x, my_y, nbr),
                device_id_type=pl.DeviceIdType.MESH,
            )
        pl.semaphore_wait(barrier_sem, 2)

        rs = [[None] * (N_Z - 1) for _ in range(2)]
        ag = [[None] * (N_Z - 1) for _ in range(2)]

        for d in (0, 1):
            rs[d][0] = rs_rdma(d, 0)
            rs[d][0].start()

        for s in range(N_Z - 1):
            for d in (0, 1):
                compute(d, s + 1)
            for d in (0, 1):
                rs[d][s].wait_recv()
                r0 = row0(d, chunk_of(d, s + 1))
                out_ref[pl.ds(r0, CHUNK), :] += rs_buf[d, s]
            if s < N_Z - 2:
                for d in (0, 1):
                    rs[d][s + 1] = rs_rdma(d, s + 1)
                    rs[d][s + 1].start()

        for s in range(N_Z - 1):
            for d in (0, 1):
                c = (my_z + 1 - s) % N_Z if d == 0 else (my_z - 1 + s) % N_Z
                r0 = row0(d, c)
                ag[d][s] = pltpu.make_async_remote_copy(
                    src_ref=out_ref.at[pl.ds(r0, CHUNK), :],
                    dst_ref=out_ref.at[pl.ds(r0, CHUNK), :],
                    send_sem=ag_send.at[d, s],
                    recv_sem=ag_recv.at[d, s],
                    device_id=(my_x, my_y, nbr_of(d)),
                    device_id_type=pl.DeviceIdType.MESH,
                )
                ag[d][s].start()
            for d in (0, 1):
                ag[d][s].wait_recv()

        for s in range(N_Z - 1):
            for d in (0, 1):
                rs[d][s].wait_send()
                ag[d][s].wait_send()

    return pl.pallas_call(
        body,
        out_shape=jax.ShapeDtypeStruct((m, n), jnp.float32),
        in_specs=[
            pl.BlockSpec(memory_space=pltpu.VMEM),
            pl.BlockSpec(memory_space=pltpu.VMEM),
        ],
        out_specs=pl.BlockSpec(memory_space=pltpu.VMEM),
        scratch_shapes=[
            pltpu.VMEM((2, N_Z - 1, CHUNK, n), jnp.float32),
            pltpu.SemaphoreType.DMA((2, N_Z - 1)),
            pltpu.SemaphoreType.DMA((2, N_Z - 1)),
            pltpu.SemaphoreType.DMA((2, N_Z - 1)),
            pltpu.SemaphoreType.DMA((2, N_Z - 1)),
        ],
        compiler_params=pltpu.CompilerParams(collective_id=0),
    )(dy, W)
